# baseline (speedup 1.0000x reference)
import functools

import jax
import jax.numpy as jnp
import numpy as np
from jax import lax
from jax.experimental import pallas as pl
from jax.experimental.pallas import tpu as pltpu
from jax.experimental.pallas import tpu_sc as plsc


def _project_body(e_ref, w_ref, b_ref, out_ref):
    p = jnp.dot(e_ref[...], w_ref[...], preferred_element_type=jnp.float32)
    out_ref[...] = p + b_ref[...]


def _project(e2, wx, bx, rblk):
    nrows, kdim = e2.shape
    return pl.pallas_call(
        _project_body,
        grid=(pl.cdiv(nrows, rblk),),
        in_specs=[
            pl.BlockSpec((rblk, kdim), lambda i: (i, 0)),
            pl.BlockSpec((kdim, 128), lambda i: (0, 0)),
            pl.BlockSpec((1, 128), lambda i: (0, 0)),
        ],
        out_specs=pl.BlockSpec((rblk, 128), lambda i: (i, 0)),
        out_shape=jax.ShapeDtypeStruct((nrows, 128), jnp.float32),
    )(e2, wx, bx)


_NBUF = 8
_L = 16


def _pool_kernel(seq_len, bpw, n_loop, pf_hbm, t2_hbm, out0_hbm, out1_hbm,
                 text_v, idx1_v, g0_v, g1_v, o0_v, o1_v, sem0, sem1):
    nc = lax.axis_size("c")
    wid = lax.axis_index("s") * nc + lax.axis_index("c")
    b0 = wid * bpw

    pltpu.sync_copy(t2_hbm.at[:, pl.ds(b0, bpw)], text_v)

    nvec = bpw // _L
    zero = jnp.zeros((_L,), jnp.float32)
    for k in range(nvec):
        o0_v[pl.ds(k * _L, _L)] = zero
        o1_v[pl.ds(k * _L, _L)] = zero

    def start(s, u):
        pltpu.async_copy(pf_hbm.at[text_v.at[s]], g0_v.at[u], sem0.at[u])
        for k in range(nvec):
            idx1_v[u, pl.ds(k * _L, _L)] = text_v[s, pl.ds(k * _L, _L)] + 1
        pltpu.async_copy(pf_hbm.at[idx1_v.at[u]], g1_v.at[u], sem1.at[u])

    def wait(u):
        pltpu.make_async_copy(pf_hbm.at[text_v.at[0]], g0_v.at[u],
                              sem0.at[u]).wait()
        pltpu.make_async_copy(pf_hbm.at[text_v.at[0]], g1_v.at[u],
                              sem1.at[u]).wait()

    def consume(u):
        for k in range(nvec):
            plsc.addupdate(o0_v.at[pl.ds(k * _L, _L)],
                           g0_v[u, pl.ds(k * _L, _L)])
            plsc.addupdate(o1_v.at[pl.ds(k * _L, _L)],
                           g1_v[u, pl.ds(k * _L, _L)])

    for u in range(_NBUF):
        start(u, u)

    def body(i, carry):
        s0 = i * _NBUF
        for u in range(_NBUF):
            wait(u)
            consume(u)
            start(s0 + _NBUF + u, u)
        return carry

    lax.fori_loop(0, n_loop, body, 0, unroll=False)

    for u in range(_NBUF):
        wait(u)
        consume(u)

    pltpu.sync_copy(o0_v, out0_hbm.at[pl.ds(b0, bpw)])
    pltpu.sync_copy(o1_v, out1_hbm.at[pl.ds(b0, bpw)])


def _pool(p_flat, t2, seq_len, batch):
    info = plsc.get_sparse_core_info()
    nw = info.num_cores * info.num_subcores
    bpw = batch // nw
    n_loop = seq_len // _NBUF - 1
    mesh = plsc.VectorSubcoreMesh(core_axis_name="c", subcore_axis_name="s")
    f = pl.kernel(
        functools.partial(_pool_kernel, seq_len, bpw, n_loop),
        out_type=[jax.ShapeDtypeStruct((batch,), jnp.float32),
                  jax.ShapeDtypeStruct((batch,), jnp.float32)],
        mesh=mesh,
        scratch_types=[
            pltpu.VMEM((seq_len, bpw), jnp.int32),
            pltpu.VMEM((_NBUF, bpw), jnp.int32),
            pltpu.VMEM((_NBUF, bpw), jnp.float32),
            pltpu.VMEM((_NBUF, bpw), jnp.float32),
            pltpu.VMEM((bpw,), jnp.float32),
            pltpu.VMEM((bpw,), jnp.float32),
            pltpu.SemaphoreType.DMA((_NBUF,)),
            pltpu.SemaphoreType.DMA((_NBUF,)),
        ],
        compiler_params=pltpu.CompilerParams(use_tc_tiling_on_sc=False,
                                             needs_layout_passes=False),
    )
    return f(p_flat, t2)


def kernel(text, embedding, W, b):
    seq_len, batch = text.shape
    vocab, embed_dim = embedding.shape
    out_dim = W.shape[1]
    grp = 128 // out_dim
    e2 = embedding.reshape(vocab // grp, grp * embed_dim)
    wx = (jnp.eye(grp, dtype=jnp.float32)[:, None, :, None]
          * W[None, :, None, :]).reshape(grp * embed_dim, 128)
    bx = jnp.tile(b, grp).reshape(1, 128)
    p = _project(e2, wx, bx, rblk=624)
    p_flat = p.reshape(vocab * out_dim) * jnp.float32(1.0 / seq_len)
    o0, o1 = _pool(p_flat, text * 2, seq_len, batch)
    return jnp.stack([o0, o1], axis=1)


# speedup vs baseline: 1.2542x; 1.2542x over previous
import functools

import jax
import jax.numpy as jnp
import numpy as np
from jax import lax
from jax.experimental import pallas as pl
from jax.experimental.pallas import tpu as pltpu
from jax.experimental.pallas import tpu_sc as plsc


def _project_body(e_ref, wt_ref, bt_ref, out_ref, *, inv_seq):
    p = lax.dot_general(wt_ref[...], e_ref[...],
                        (((1,), (1,)), ((), ())),
                        preferred_element_type=jnp.float32)
    out_ref[...] = (p + bt_ref[...]) * inv_seq


def _project(embedding, Wt, bt, seq_len, blk):
    vocab, embed_dim = embedding.shape
    out_dim = Wt.shape[0]
    return pl.pallas_call(
        functools.partial(_project_body, inv_seq=1.0 / seq_len),
        grid=(pl.cdiv(vocab, blk),),
        in_specs=[
            pl.BlockSpec((blk, embed_dim), lambda i: (i, 0)),
            pl.BlockSpec((out_dim, embed_dim), lambda i: (0, 0)),
            pl.BlockSpec((out_dim, 1), lambda i: (0, 0)),
        ],
        out_specs=pl.BlockSpec((out_dim, blk), lambda i: (0, i)),
        out_shape=jax.ShapeDtypeStruct((out_dim, vocab), jnp.float32),
    )(embedding, Wt, bt)


_NBUF = 8
_L = 16


def _pool_kernel(seq_len, bpw, n_loop, p0_hbm, p1_hbm, text_hbm,
                 out0_hbm, out1_hbm, text_v, g0_v, g1_v, o0_v, o1_v,
                 sem0, sem1):
    nc = lax.axis_size("c")
    wid = lax.axis_index("s") * nc + lax.axis_index("c")
    b0 = wid * bpw

    pltpu.sync_copy(text_hbm.at[:, pl.ds(b0, bpw)], text_v)

    nvec = bpw // _L
    zero = jnp.zeros((_L,), jnp.float32)
    for k in range(nvec):
        o0_v[pl.ds(k * _L, _L)] = zero
        o1_v[pl.ds(k * _L, _L)] = zero

    def start(s, u):
        pltpu.async_copy(p0_hbm.at[text_v.at[s]], g0_v.at[u], sem0.at[u])
        pltpu.async_copy(p1_hbm.at[text_v.at[s]], g1_v.at[u], sem1.at[u])

    def wait(u):
        pltpu.make_async_copy(p0_hbm.at[text_v.at[0]], g0_v.at[u],
                              sem0.at[u]).wait()
        pltpu.make_async_copy(p1_hbm.at[text_v.at[0]], g1_v.at[u],
                              sem1.at[u]).wait()

    def consume(u):
        for k in range(nvec):
            plsc.addupdate(o0_v.at[pl.ds(k * _L, _L)],
                           g0_v[u, pl.ds(k * _L, _L)])
            plsc.addupdate(o1_v.at[pl.ds(k * _L, _L)],
                           g1_v[u, pl.ds(k * _L, _L)])

    for u in range(_NBUF):
        start(u, u)

    def body(i, carry):
        s0 = i * _NBUF
        for u in range(_NBUF):
            wait(u)
            consume(u)
            start(s0 + _NBUF + u, u)
        return carry

    lax.fori_loop(0, n_loop, body, 0, unroll=False)

    for u in range(_NBUF):
        wait(u)
        consume(u)

    pltpu.sync_copy(o0_v, out0_hbm.at[pl.ds(b0, bpw)])
    pltpu.sync_copy(o1_v, out1_hbm.at[pl.ds(b0, bpw)])


def _pool(p0, p1, text, seq_len, batch):
    info = plsc.get_sparse_core_info()
    nw = info.num_cores * info.num_subcores
    bpw = batch // nw
    n_loop = seq_len // _NBUF - 1
    mesh = plsc.VectorSubcoreMesh(core_axis_name="c", subcore_axis_name="s")
    f = pl.kernel(
        functools.partial(_pool_kernel, seq_len, bpw, n_loop),
        out_type=[jax.ShapeDtypeStruct((batch,), jnp.float32),
                  jax.ShapeDtypeStruct((batch,), jnp.float32)],
        mesh=mesh,
        scratch_types=[
            pltpu.VMEM((seq_len, bpw), jnp.int32),
            pltpu.VMEM((_NBUF, bpw), jnp.float32),
            pltpu.VMEM((_NBUF, bpw), jnp.float32),
            pltpu.VMEM((bpw,), jnp.float32),
            pltpu.VMEM((bpw,), jnp.float32),
            pltpu.SemaphoreType.DMA((_NBUF,)),
            pltpu.SemaphoreType.DMA((_NBUF,)),
        ],
        compiler_params=pltpu.CompilerParams(use_tc_tiling_on_sc=False,
                                             needs_layout_passes=False),
    )
    return f(p0, p1, text)


def kernel(text, embedding, W, b):
    seq_len, batch = text.shape
    pt = _project(embedding, W.T, b.reshape(-1, 1), seq_len, blk=16384)
    o0, o1 = _pool(pt[0], pt[1], text, seq_len, batch)
    return jnp.stack([o0, o1], axis=1)
